# Initial kernel scaffold; baseline (speedup 1.0000x reference)
import jax
import jax.numpy as jnp
from jax.experimental import pallas as pl


def kernel(x, centroid_emb):
    raise NotImplementedError("write your pallas kernel here")


# speedup vs baseline: 1.2727x; 1.2727x over previous
import functools

import jax
import jax.numpy as jnp
from jax.experimental import pallas as pl

_EPS = 1e-12


def _centroid_kernel(x_ref, c_ref, o_ref):
    xb = x_ref[...]
    c = c_ref[...]

    xn = xb * jax.lax.rsqrt(
        jnp.maximum(jnp.sum(xb * xb, axis=1, keepdims=True), _EPS * _EPS)
    )
    cn = c * jax.lax.rsqrt(
        jnp.maximum(jnp.sum(c * c, axis=1, keepdims=True), _EPS * _EPS)
    )

    sims = jax.lax.dot_general(
        xn, cn, (((1,), (1,)), ((), ())), preferred_element_type=jnp.float32
    )

    m = jnp.max(sims, axis=1, keepdims=True)
    e = jnp.exp(sims - m)
    s = jnp.sum(e, axis=1, keepdims=True)
    ctx = jnp.dot(e, c, preferred_element_type=jnp.float32)
    o_ref[...] = ctx / s


@functools.partial(jax.jit, static_argnames=("block_n",))
def _centroid_layer(x, centroid_emb, block_n=512):
    n, d = x.shape
    p, _ = centroid_emb.shape
    return pl.pallas_call(
        _centroid_kernel,
        grid=(n // block_n,),
        in_specs=[
            pl.BlockSpec((block_n, d), lambda i: (i, 0)),
            pl.BlockSpec((p, d), lambda i: (0, 0)),
        ],
        out_specs=pl.BlockSpec((block_n, d), lambda i: (i, 0)),
        out_shape=jax.ShapeDtypeStruct((n, d), jnp.float32),
    )(x, centroid_emb)


def kernel(x, centroid_emb):
    return _centroid_layer(x, centroid_emb)


# speedup vs baseline: 1.8356x; 1.4422x over previous
import functools

import jax
import jax.numpy as jnp
from jax.experimental import pallas as pl

_EPS = 1e-12


def _centroid_kernel(x_ref, c_ref, o_ref):
    xb = x_ref[...]
    c = c_ref[...]

    xn = xb * jax.lax.rsqrt(
        jnp.maximum(jnp.sum(xb * xb, axis=1, keepdims=True), _EPS * _EPS)
    )
    cn = c * jax.lax.rsqrt(
        jnp.maximum(jnp.sum(c * c, axis=1, keepdims=True), _EPS * _EPS)
    )

    sims = jax.lax.dot_general(
        xn.astype(jnp.bfloat16),
        cn.astype(jnp.bfloat16),
        (((1,), (1,)), ((), ())),
        preferred_element_type=jnp.float32,
    )

    e = jnp.exp(sims)
    s = jnp.sum(e, axis=1, keepdims=True)
    ctx = jnp.dot(
        e.astype(jnp.bfloat16), c.astype(jnp.bfloat16),
        preferred_element_type=jnp.float32,
    )
    o_ref[...] = ctx / s


@functools.partial(jax.jit, static_argnames=("block_n",))
def _centroid_layer(x, centroid_emb, block_n=512):
    n, d = x.shape
    p, _ = centroid_emb.shape
    return pl.pallas_call(
        _centroid_kernel,
        grid=(n // block_n,),
        in_specs=[
            pl.BlockSpec((block_n, d), lambda i: (i, 0)),
            pl.BlockSpec((p, d), lambda i: (0, 0)),
        ],
        out_specs=pl.BlockSpec((block_n, d), lambda i: (i, 0)),
        out_shape=jax.ShapeDtypeStruct((n, d), jnp.float32),
    )(x, centroid_emb)


def kernel(x, centroid_emb):
    return _centroid_layer(x, centroid_emb)


# speedup vs baseline: 1.8682x; 1.0178x over previous
import functools

import jax
import jax.numpy as jnp
from jax.experimental import pallas as pl
from jax.experimental.pallas import tpu as pltpu

_EPS = 1e-12


def _centroid_kernel(x_ref, c_ref, o_ref, cn_ref, cb_ref):
    @pl.when(pl.program_id(0) == 0)
    def _():
        c = c_ref[...]
        cn = c * jax.lax.rsqrt(
            jnp.maximum(jnp.sum(c * c, axis=1, keepdims=True), _EPS * _EPS)
        )
        cn_ref[...] = cn.astype(jnp.bfloat16)
        cb_ref[...] = c.astype(jnp.bfloat16)

    xb = x_ref[...]
    xn = xb * jax.lax.rsqrt(
        jnp.maximum(jnp.sum(xb * xb, axis=1, keepdims=True), _EPS * _EPS)
    )

    sims = jax.lax.dot_general(
        xn.astype(jnp.bfloat16),
        cn_ref[...],
        (((1,), (1,)), ((), ())),
        preferred_element_type=jnp.float32,
    )
    e = jnp.exp(sims.astype(jnp.bfloat16))
    s = jnp.sum(e.astype(jnp.float32), axis=1, keepdims=True)
    ctx = jnp.dot(e, cb_ref[...], preferred_element_type=jnp.float32)
    o_ref[...] = ctx / s


@functools.partial(jax.jit, static_argnames=("block_n",))
def _centroid_layer(x, centroid_emb, block_n=512):
    n, d = x.shape
    p, _ = centroid_emb.shape
    return pl.pallas_call(
        _centroid_kernel,
        grid=(n // block_n,),
        in_specs=[
            pl.BlockSpec((block_n, d), lambda i: (i, 0)),
            pl.BlockSpec((p, d), lambda i: (0, 0)),
        ],
        out_specs=pl.BlockSpec((block_n, d), lambda i: (i, 0)),
        out_shape=jax.ShapeDtypeStruct((n, d), jnp.float32),
        scratch_shapes=[
            pltpu.VMEM((p, d), jnp.bfloat16),
            pltpu.VMEM((p, d), jnp.bfloat16),
        ],
    )(x, centroid_emb)


def kernel(x, centroid_emb):
    return _centroid_layer(x, centroid_emb)
